# baseline (speedup 1.0000x reference)
import jax
import jax.numpy as jnp
from jax.experimental import pallas as pl
from jax.experimental.pallas import tpu as pltpu

N = 16
E = 64
H = 128

_CONVW_OFF = [0, 48, 176, 304]
_LINW_OFF = 432
_SMALL_OFF = 560
_POOL_OFF = 568
_R128 = 576

_R256 = 264 * 4


def _net_kernel(x_ref, ei_ref, p128_hbm, p256_hbm, out_ref,
                v128, v256, sem):
    f32 = jnp.float32

    pltpu.make_async_copy(p128_hbm, v128, sem.at[0]).start()
    fc_copies = []
    for i in range(4):
        c = pltpu.make_async_copy(
            p256_hbm.at[pl.ds(264 * i, 264)],
            v256.at[pl.ds(264 * i, 264)], sem.at[1 + i])
        c.start()
        fc_copies.append(c)

    def dotT(a, b):
        return jax.lax.dot_general(a, b, (((0,), (0,)), ((), ())),
                                   preferred_element_type=f32)

    def mm(a, b):
        return jax.lax.dot_general(a, b, (((1,), (0,)), ((), ())),
                                   preferred_element_type=f32)

    def rowdot(a, w_row):
        return jnp.sum(a * w_row, axis=1, keepdims=True)

    srcT = ei_ref[0:1, :]
    dstT = ei_ref[1:2, :]
    rowN = jax.lax.broadcasted_iota(jnp.int32, (N, E), 0)
    ST = (srcT == rowN).astype(f32)
    DT = (dstT == rowN).astype(f32)
    mask = jnp.ones((1, E), dtype=f32)

    row_i = jax.lax.broadcasted_iota(jnp.int32, (N, N), 0)
    col_i = jax.lax.broadcasted_iota(jnp.int32, (N, N), 1)
    eye = (row_i == col_i).astype(f32)
    colf = col_i.astype(f32)
    valid_col = jax.lax.broadcasted_iota(jnp.int32, (N, 1), 0)

    x = x_ref[:, :]

    pltpu.make_async_copy(p128_hbm, v128, sem.at[0]).wait()
    smalls = v128[pl.ds(_SMALL_OFF, 8), :]
    pools = v128[pl.ds(_POOL_OFF, 8), :]

    n_cur = N
    for l in range(4):
        rows = 48 if l == 0 else H
        W = v128[pl.ds(_CONVW_OFF[l], rows), :]
        b = smalls[l:l + 1, :]
        brel = smalls[5:6, l:l + 1]
        wrel = pools[l:l + 1, :]
        wroot = pools[4 + l:5 + l, :]

        xw = mm(x, W)
        deg = jnp.sum(DT * mask, axis=1, keepdims=True) + 1.0
        dinv = 1.0 / jnp.sqrt(deg)
        norm = mask * dotT(dinv, ST) * dotT(dinv, DT)
        gath = dotT(ST, xw)
        aggc = mm(DT * norm, gath)
        x = jax.nn.relu(aggc + (1.0 / deg) * xw + b)

        agg2 = mm(DT * mask, dotT(ST, x))
        raw = rowdot(agg2, wrel) + brel + rowdot(x, wroot)
        score = jnp.tanh(raw)
        score = jnp.where(valid_col < n_cur, score, -2.0)

        k = (n_cur + 1) // 2
        s_row = dotT(score, eye)
        s_cb = jax.lax.broadcast_in_dim(score, (N, N), (0, 1))
        s_rb = jax.lax.broadcast_in_dim(s_row, (N, N), (0, 1))
        beats = (s_rb > s_cb) | ((s_rb == s_cb) & (col_i < row_i))
        rank = jnp.sum(beats.astype(f32), axis=1, keepdims=True)
        PT = ((rank == colf) & (colf < float(k))).astype(f32)

        sel_score = dotT(PT, score)
        x = dotT(PT, x) * sel_score
        ST = dotT(PT, ST)
        DT = dotT(PT, DT)
        mask = (mask * jnp.sum(ST, axis=0, keepdims=True)
                     * jnp.sum(DT, axis=0, keepdims=True))
        n_cur = k

    lin_W = v128[pl.ds(_LINW_OFF, H), :]
    lin_b = smalls[4:5, :]
    out2 = mm(x[0:1, :], lin_W) + lin_b
    m = jnp.max(out2, axis=1, keepdims=True)
    z = out2 - m
    out2 = z - jnp.log(jnp.sum(jnp.exp(z), axis=1, keepdims=True))

    h = jnp.concatenate([jnp.zeros((1, H), dtype=f32), out2], axis=1)

    for l in range(4):
        fc_copies[l].wait()
        off = 264 * l
        fcW = v256[pl.ds(off, 256), :]
        grp = v256[pl.ds(off + 256, 8), :]
        h = mm(h, fcW) + grp[0:1, :]
        if l < 3:
            mu = jnp.mean(h, axis=1, keepdims=True)
            var = jnp.mean((h - mu) ** 2, axis=1, keepdims=True)
            h = (h - mu) / jnp.sqrt(var + 1e-5) * grp[1:2, :] + grp[2:3, :]
            h = jax.nn.relu(h)

    out_ref[:, :] = h


def kernel(sp_x, sp_edge_index, params):
    f32 = jnp.float32
    p = params

    ei = jnp.zeros((8, E), dtype=jnp.int32).at[:2, :].set(
        sp_edge_index.astype(jnp.int32))
    x48 = jnp.zeros((N, 48), dtype=f32).at[:, :45].set(sp_x)

    parts = [p['conv0_W'].reshape(-1), jnp.zeros((3 * H,), f32)]
    for l in range(1, 4):
        parts.append(p['conv%d_W' % l].reshape(-1))
    parts.append(p['lin_W'].reshape(-1))
    for l in range(4):
        parts.append(p['conv%d_b' % l].reshape(-1))
    parts.append(p['lin_b'].reshape(-1))
    brels = jnp.stack([p['pool%d_brel' % l][0] for l in range(4)])
    parts += [brels, jnp.zeros((124,), f32), jnp.zeros((2 * H,), f32)]
    for l in range(4):
        parts.append(p['pool%d_Wrel' % l].reshape(-1))
    for l in range(4):
        parts.append(p['pool%d_Wroot' % l].reshape(-1))
    pack128 = jnp.concatenate(parts).reshape(_R128, H)

    parts = []
    for l in range(3):
        parts += [
            p['fc%d_W' % l].reshape(-1),
            p['fc%d_b' % l].reshape(-1),
            p['ln%d_w' % l].reshape(-1),
            p['ln%d_b' % l].reshape(-1),
            jnp.zeros((5 * 256,), f32),
        ]
    parts += [p['fc3_W'].reshape(-1), p['fc3_b'].reshape(-1),
              jnp.zeros((7 * 256,), f32)]
    pack256 = jnp.concatenate(parts).reshape(_R256, 256)

    out = pl.pallas_call(
        _net_kernel,
        out_shape=jax.ShapeDtypeStruct((1, 256), f32),
        in_specs=[pl.BlockSpec(memory_space=pltpu.MemorySpace.VMEM)] * 2
                 + [pl.BlockSpec(memory_space=pltpu.MemorySpace.HBM)] * 2,
        out_specs=pl.BlockSpec(memory_space=pltpu.MemorySpace.VMEM),
        scratch_shapes=[
            pltpu.VMEM((_R128, H), f32),
            pltpu.VMEM((_R256, 256), f32),
            pltpu.SemaphoreType.DMA((5,)),
        ],
    )(x48, ei, pack128, pack256)
    return out.reshape(-1)


# speedup vs baseline: 2.6249x; 2.6249x over previous
import jax
import jax.numpy as jnp
from jax.experimental import pallas as pl
from jax.experimental.pallas import tpu as pltpu

N = 16
E = 64
H = 128

_RS = 40


def _net_kernel(x_ref, ei_ref, sp_ref,
                w0_ref, w1_ref, w2_ref, w3_ref, lin_ref,
                f0_ref, f1_ref, f2_ref, f3_ref, out_ref):
    f32 = jnp.float32
    w_refs = (w0_ref, w1_ref, w2_ref, w3_ref)
    f_refs = (f0_ref, f1_ref, f2_ref, f3_ref)

    def dotT(a, b):
        return jax.lax.dot_general(a, b, (((0,), (0,)), ((), ())),
                                   preferred_element_type=f32)

    def mm(a, b):
        return jax.lax.dot_general(a, b, (((1,), (0,)), ((), ())),
                                   preferred_element_type=f32)

    def rowdot(a, w_row):
        return jnp.sum(a * w_row, axis=1, keepdims=True)

    sp = sp_ref[:, :]

    srcT = ei_ref[0:1, :]
    dstT = ei_ref[1:2, :]
    rowN = jax.lax.broadcasted_iota(jnp.int32, (N, E), 0)
    ST = (srcT == rowN).astype(f32)
    DT = (dstT == rowN).astype(f32)
    mask = jnp.ones((1, E), dtype=f32)

    row_i = jax.lax.broadcasted_iota(jnp.int32, (N, N), 0)
    col_i = jax.lax.broadcasted_iota(jnp.int32, (N, N), 1)
    eye = (row_i == col_i).astype(f32)
    colf = col_i.astype(f32)
    valid_col = jax.lax.broadcasted_iota(jnp.int32, (N, 1), 0)

    S = dotT(ST, eye)
    x = x_ref[:, :]

    n_cur = N
    for l in range(4):
        W = w_refs[l][:, :]
        b = sp[l:l + 1, :]
        wrel = sp[4 + l:5 + l, :]
        wroot = sp[8 + l:9 + l, :]
        brel = sp[33:34, l:l + 1]

        DTm = DT * mask
        deg = jnp.sum(DTm, axis=1, keepdims=True) + 1.0
        dinv = 1.0 / jnp.sqrt(deg)
        norm = mask * dotT(dinv, ST) * dotT(dinv, DT)
        M = mm(DT * norm, S) + eye * (1.0 / deg)
        xw = mm(x, W)
        x = jax.nn.relu(mm(M, xw) + b)

        B = mm(DTm, S)
        raw = mm(B, rowdot(x, wrel)) + rowdot(x, wroot) + brel
        score = jnp.tanh(raw)
        score = jnp.where(valid_col < n_cur, score, -2.0)

        k = (n_cur + 1) // 2
        s_row = dotT(score, eye)
        s_cb = jax.lax.broadcast_in_dim(score, (N, N), (0, 1))
        s_rb = jax.lax.broadcast_in_dim(s_row, (N, N), (0, 1))
        beats = (s_rb > s_cb) | ((s_rb == s_cb) & (col_i < row_i))
        rank = jnp.sum(beats.astype(f32), axis=1, keepdims=True)
        PT = ((rank == colf) & (colf < float(k))).astype(f32)

        sel_score = dotT(PT, score)
        x = dotT(PT, x) * sel_score
        S = mm(S, PT)
        ST = dotT(PT, ST)
        DT = dotT(PT, DT)
        mask = (mask * jnp.sum(ST, axis=0, keepdims=True)
                     * jnp.sum(DT, axis=0, keepdims=True))
        n_cur = k

    out2 = mm(x[0:1, :], lin_ref[:, :]) + sp[12:13, :]
    m = jnp.max(out2, axis=1, keepdims=True)
    z = out2 - m
    out2 = z - jnp.log(jnp.sum(jnp.exp(z), axis=1, keepdims=True))

    h = jnp.concatenate([jnp.zeros((1, H), dtype=f32), out2], axis=1)

    def row256(r):
        return jnp.concatenate([sp[r:r + 1, :], sp[r + 1:r + 2, :]], axis=1)

    for l in range(4):
        h = mm(h, f_refs[l][:, :])
        if l < 3:
            h = h + row256(13 + 6 * l)
            mu = jnp.mean(h, axis=1, keepdims=True)
            var = jnp.mean((h - mu) ** 2, axis=1, keepdims=True)
            h = ((h - mu) / jnp.sqrt(var + 1e-5) * row256(15 + 6 * l)
                 + row256(17 + 6 * l))
            h = jax.nn.relu(h)
        else:
            h = h + row256(31)

    out_ref[:, :] = h


def kernel(sp_x, sp_edge_index, params):
    f32 = jnp.float32
    p = params

    ei = jnp.zeros((8, E), dtype=jnp.int32).at[:2, :].set(
        sp_edge_index.astype(jnp.int32))

    parts = []
    for l in range(4):
        parts.append(p['conv%d_b' % l].reshape(-1))
    for l in range(4):
        parts.append(p['pool%d_Wrel' % l].reshape(-1))
    for l in range(4):
        parts.append(p['pool%d_Wroot' % l].reshape(-1))
    parts.append(p['lin_b'].reshape(-1))
    for l in range(3):
        parts += [p['fc%d_b' % l].reshape(-1),
                  p['ln%d_w' % l].reshape(-1),
                  p['ln%d_b' % l].reshape(-1)]
    parts.append(p['fc3_b'].reshape(-1))
    for l in range(4):
        parts.append(p['pool%d_brel' % l])
    parts.append(jnp.zeros((124 + 6 * H,), f32))
    smallpack = jnp.concatenate(parts).reshape(_RS, H)

    out = pl.pallas_call(
        _net_kernel,
        out_shape=jax.ShapeDtypeStruct((1, 256), f32),
    )(sp_x, ei, smallpack,
      p['conv0_W'], p['conv1_W'], p['conv2_W'], p['conv3_W'], p['lin_W'],
      p['fc0_W'], p['fc1_W'], p['fc2_W'], p['fc3_W'])
    return out.reshape(-1)
